# baseline (speedup 1.0000x reference)
import functools
import math

import jax
import jax.numpy as jnp
import numpy as np
from jax import lax
from jax.experimental import pallas as pl
from jax.experimental.pallas import tpu as pltpu
from jax.experimental.pallas import tpu_sc as plsc

L = 16
T = 2 ** 19
F = 2
N_MIN = 2 ** 4
BOUND = 2.0
N_MAX = int(2 ** 11 * BOUND)
N_PTS = 524288
ENC_DIM = L * F
HIDDEN = 64
MASK = T - 1
LT = L * T

_B = math.exp((math.log(N_MAX) - math.log(N_MIN)) / (L - 1))
RES = [int(math.floor(N_MIN * (_B ** l))) for l in range(L)]
P2 = int(np.uint32(2654435761).astype(np.int32))
P3 = int(np.uint32(805459861).astype(np.int32))

NC = 2
NS = 16
NW = NC * NS
PTS_W = N_PTS // NW
C = 512
G = C // 16
CHUNKS = PTS_W // C
NIDX = 8 * C

IK = 2048
IROWS_W = LT // NW
ICHUNKS = IROWS_W // IK


def _sc_interleave(tp, tm):
    mesh = plsc.VectorSubcoreMesh(
        core_axis_name="c", subcore_axis_name="s", num_cores=NC,
        num_subcores=NS)

    @functools.partial(
        pl.kernel,
        out_type=jax.ShapeDtypeStruct((LT, 4), jnp.float32),
        mesh=mesh,
        compiler_params=pltpu.CompilerParams(
            needs_layout_passes=False, use_tc_tiling_on_sc=False),
        scratch_types=[
            pltpu.VMEM((IK, 2), jnp.float32),
            pltpu.VMEM((IK, 2), jnp.float32),
            pltpu.VMEM((IK, 4), jnp.float32),
        ],
    )
    def ikernel(tp_hbm, tm_hbm, out_hbm, tpc, tmc, oc):
        wid = lax.axis_index("s") * NC + lax.axis_index("c")
        iota = lax.iota(jnp.int32, 16)
        col0 = jnp.full((16,), 0, jnp.int32)
        col1 = col0 + 1

        def chunk_body(ci, carry):
            base = pl.multiple_of(wid * IROWS_W + ci * IK, IK)
            pltpu.sync_copy(tp_hbm.at[pl.ds(base, IK)], tpc)
            pltpu.sync_copy(tm_hbm.at[pl.ds(base, IK)], tmc)

            def body(g, carry2):
                rows = g * 16 + iota
                p0 = plsc.load_gather(tpc, [rows, col0])
                p1 = plsc.load_gather(tpc, [rows, col1])
                m0 = plsc.load_gather(tmc, [rows, col0])
                m1 = plsc.load_gather(tmc, [rows, col1])
                plsc.store_scatter(oc, [rows, col0], p0)
                plsc.store_scatter(oc, [rows, col1], p1)
                plsc.store_scatter(oc, [rows, col1 + 1], m0)
                plsc.store_scatter(oc, [rows, col1 + 2], m1)
                return carry2

            lax.fori_loop(0, IK // 16, body, 0)
            pltpu.sync_copy(oc, out_hbm.at[pl.ds(base, IK)])
            return carry

        lax.fori_loop(0, ICHUNKS, chunk_body, 0)

    return ikernel(tp, tm)


def _sc_encode(r, phi, ctab):
    mesh = plsc.VectorSubcoreMesh(
        core_axis_name="c", subcore_axis_name="s", num_cores=NC,
        num_subcores=NS)

    @functools.partial(
        pl.kernel,
        out_type=jax.ShapeDtypeStruct((N_PTS, ENC_DIM), jnp.float32),
        mesh=mesh,
        compiler_params=pltpu.CompilerParams(
            needs_layout_passes=False, use_tc_tiling_on_sc=False),
        scratch_types=[
            pltpu.VMEM((C, 3), jnp.float32),
            pltpu.VMEM((C,), jnp.float32),
            pltpu.VMEM((C,), jnp.float32),
            pltpu.VMEM((C,), jnp.float32),
            pltpu.VMEM((C,), jnp.float32),
            pltpu.VMEM((C,), jnp.int32),
            pltpu.VMEM((NIDX,), jnp.int32),
            pltpu.VMEM((NIDX,), jnp.int32),
            pltpu.VMEM((NIDX, 4), jnp.float32),
            pltpu.VMEM((NIDX, 4), jnp.float32),
            pltpu.VMEM((C, ENC_DIM), jnp.float32),
            pltpu.SemaphoreType.DMA,
            pltpu.SemaphoreType.DMA,
        ],
    )
    def enc_kernel(r_hbm, phi_hbm, ctab_hbm, out_hbm, rc, phc, xs, ys, zs,
                   csel, idxa, idxb, rowsa, rowsb, enc_c, sema, semb):
        wid = lax.axis_index("s") * NC + lax.axis_index("c")
        iota = lax.iota(jnp.int32, 16)
        zeros16 = jnp.zeros((16,), jnp.float32)
        col0 = jnp.full((16,), 0, jnp.int32)
        col1 = col0 + 1
        idxbufs = (idxa, idxb)
        rowbufs = (rowsa, rowsb)
        sems = (sema, semb)

        def compute_idx(l, dst):
            rf = float(RES[l])
            lbase = l * T

            def body(g, carry):
                s = pl.ds(g * 16, 16)
                cx0 = (xs[s] * rf).astype(jnp.int32)
                cy0 = (ys[s] * rf).astype(jnp.int32)
                cz0 = (zs[s] * rf).astype(jnp.int32)
                hy0 = cy0 * P2
                hz0 = cz0 * P3
                hy1 = hy0 + P2
                hz1 = hz0 + P3
                cx1 = cx0 + 1
                t = (hy0 ^ hz0, hy0 ^ hz1, hy1 ^ hz0, hy1 ^ hz1)
                j = 0
                for cxv in (cx0, cx1):
                    for tyz in t:
                        dst[pl.ds(j * C + g * 16, 16)] = (
                            ((cxv ^ tyz) & MASK) + lbase)
                        j += 1
                return carry

            lax.fori_loop(0, G, body, 0)

        def fire(l):
            compute_idx(l, idxbufs[l % 2])
            return pltpu.async_copy(
                ctab_hbm.at[idxbufs[l % 2]], rowbufs[l % 2], sems[l % 2])

        def accumulate(l, rows):
            rf = float(RES[l])

            def body(g, carry):
                s = pl.ds(g * 16, 16)
                px = xs[s] * rf
                py = ys[s] * rf
                pz = zs[s] * rf
                fx = px - px.astype(jnp.int32).astype(jnp.float32)
                fy = py - py.astype(jnp.int32).astype(jnp.float32)
                fz = pz - pz.astype(jnp.int32).astype(jnp.float32)
                gx = 1.0 - fx
                gy = 1.0 - fy
                gz = 1.0 - fz
                cs = csel[s]
                cs1 = cs + 1
                rbase = g * 16 + iota
                wxy = (gx * gy, gx * fy, fx * gy, fx * fy)
                acc0 = zeros16
                acc1 = zeros16
                j = 0
                for i in range(4):
                    for wz in (gz, fz):
                        w = wxy[i] * wz
                        rv = rbase + (j * C)
                        acc0 = acc0 + w * plsc.load_gather(rows, [rv, cs])
                        acc1 = acc1 + w * plsc.load_gather(rows, [rv, cs1])
                        j += 1
                enc_col = jnp.full((16,), 2 * l, jnp.int32)
                plsc.store_scatter(enc_c, [rbase, enc_col], acc0)
                plsc.store_scatter(enc_c, [rbase, enc_col + 1], acc1)
                return carry

            lax.fori_loop(0, G, body, 0)

        def chunk_body(ci, carry):
            base = pl.multiple_of(wid * PTS_W + ci * C, C)
            pltpu.sync_copy(r_hbm.at[pl.ds(base, C)], rc)
            pltpu.sync_copy(phi_hbm.at[pl.ds(base, C)], phc)

            def prep(g, carry2):
                rbase = g * 16 + iota
                x = plsc.load_gather(rc, [rbase, col0])
                y = plsc.load_gather(rc, [rbase, col1])
                z = plsc.load_gather(rc, [rbase, col1 + 1])
                s = pl.ds(g * 16, 16)
                xs[s] = jnp.minimum(jnp.maximum((x + 2.0) * 0.25, 0.0), 1.0)
                ys[s] = jnp.minimum(jnp.maximum((y + 2.0) * 0.25, 0.0), 1.0)
                zs[s] = jnp.minimum(jnp.maximum((z + 2.0) * 0.25, 0.0), 1.0)
                csel[s] = jnp.where(phc[s] >= 0.0, col0, col0 + 2)
                return carry2

            lax.fori_loop(0, G, prep, 0)

            descs = {0: fire(0)}
            for l in range(L):
                if l + 1 < L:
                    descs[l + 1] = fire(l + 1)
                descs[l].wait()
                accumulate(l, rowbufs[l % 2])
            pltpu.sync_copy(enc_c, out_hbm.at[pl.ds(base, C)])
            return carry

        lax.fori_loop(0, CHUNKS, chunk_body, 0)

    return enc_kernel(r, phi, ctab)


_BN = 4096


def _mlp_body(enc_ref, phi_ref, w1p_ref, b1p_ref, w2p_ref, b2p_ref,
              w1m_ref, b1m_ref, w2m_ref, b2m_ref, out_ref):
    e = enc_ref[...]
    hp = jnp.maximum(
        jnp.dot(e, w1p_ref[...], preferred_element_type=jnp.float32)
        + b1p_ref[...], 0.0)
    sp = jnp.dot(hp, w2p_ref[...], preferred_element_type=jnp.float32) \
        + b2p_ref[...]
    hm = jnp.maximum(
        jnp.dot(e, w1m_ref[...], preferred_element_type=jnp.float32)
        + b1m_ref[...], 0.0)
    sm = jnp.dot(hm, w2m_ref[...], preferred_element_type=jnp.float32) \
        + b2m_ref[...]
    out_ref[...] = jnp.where(phi_ref[...] >= 0.0, sp, sm)


def _tc_mlp(enc, phi_r, W1p, b1p, W2p, b2p, W1m, b1m, W2m, b2m):
    grid = N_PTS // _BN
    full = lambda shape: pl.BlockSpec(shape, lambda i: (0, 0))
    return pl.pallas_call(
        _mlp_body,
        grid=(grid,),
        in_specs=[
            pl.BlockSpec((_BN, ENC_DIM), lambda i: (i, 0)),
            pl.BlockSpec((_BN, 1), lambda i: (i, 0)),
            full((ENC_DIM, HIDDEN)), full((1, HIDDEN)),
            full((HIDDEN, 1)), full((1, 1)),
            full((ENC_DIM, HIDDEN)), full((1, HIDDEN)),
            full((HIDDEN, 1)), full((1, 1)),
        ],
        out_specs=pl.BlockSpec((_BN, 1), lambda i: (i, 0)),
        out_shape=jax.ShapeDtypeStruct((N_PTS, 1), jnp.float32),
    )(enc, phi_r, W1p, b1p.reshape(1, HIDDEN), W2p, b2p.reshape(1, 1),
      W1m, b1m.reshape(1, HIDDEN), W2m, b2m.reshape(1, 1))


def kernel(r, phi_r, table_p, W1p, b1p, W2p, b2p, table_m, W1m, b1m, W2m,
           b2m):
    ctab = _sc_interleave(table_p.reshape(LT, F), table_m.reshape(LT, F))
    enc = _sc_encode(r, phi_r.reshape(N_PTS), ctab)
    return _tc_mlp(enc, phi_r, W1p, b1p, W2p, b2p, W1m, b1m, W2m, b2m)
